# Initial kernel scaffold; baseline (speedup 1.0000x reference)
import jax
import jax.numpy as jnp
from jax.experimental import pallas as pl


def kernel(x):
    raise NotImplementedError("write your pallas kernel here")


# speedup vs baseline: 54.2827x; 54.2827x over previous
import functools

import jax
import jax.numpy as jnp
from jax.experimental import pallas as pl
from jax.experimental.pallas import tpu as pltpu

_B, _C, _H, _W = 4, 64, 384, 384
_HB = 48
_NH = _H // _HB
_SPLITS = (16, 32, 48)
_K = 2 * len(_SPLITS)


def _sums_body(x_ref, s_ref):
    h = pl.program_id(1)
    part = jnp.sum(x_ref[0], axis=(1, 2))

    @pl.when(h == 0)
    def _():
        s_ref[0, 0, :] = part

    @pl.when(h != 0)
    def _():
        s_ref[0, 0, :] = s_ref[0, 0, :] + part


def _weights_body(s_ref, w_ref):
    m = s_ref[:, 0, :]
    mc = m[:, :, None]
    mj = m[:, None, :]
    jj = jax.lax.broadcasted_iota(jnp.int32, (_B, _C, _C), 2)
    cc = jax.lax.broadcasted_iota(jnp.int32, (_B, _C, _C), 1)
    cnt = jnp.where((mj < mc) | ((mj == mc) & (jj < cc)), 1, 0)
    rank = jnp.sum(cnt, axis=2)
    ci = jax.lax.broadcasted_iota(jnp.int32, (_B, _C), 1)
    ws = []
    for sb in _SPLITS:
        ws.append(jnp.where(rank < sb, -1.0 / sb, 1.0 / (_C - sb)))
    for sb in _SPLITS:
        ws.append(jnp.where(ci < sb, -1.0 / sb, 1.0 / (_C - sb)))
    w_ref[...] = jnp.stack(ws, axis=1)


def _rcp_body(x_ref, w_ref, y_ref, mn_ref, mx_ref):
    h = pl.program_id(1)
    xb = x_ref[0]
    wb = w_ref[0]
    accs = []
    for k in range(_K):
        acc = jnp.zeros((_HB, _W), jnp.float32)
        for c in range(_C):
            acc = acc + wb[k, c] * xb[c]
        accs.append(acc)
    y = jnp.stack(accs)
    y_ref[0] = y
    vmin = jnp.min(y, axis=(1, 2))
    vmax = jnp.max(y, axis=(1, 2))

    @pl.when(h == 0)
    def _():
        mn_ref[0, 0, :] = vmin
        mx_ref[0, 0, :] = vmax

    @pl.when(h != 0)
    def _():
        mn_ref[0, 0, :] = jnp.minimum(mn_ref[0, 0, :], vmin)
        mx_ref[0, 0, :] = jnp.maximum(mx_ref[0, 0, :], vmax)


def _norm_body(y_ref, mn_ref, mx_ref, o_ref):
    yb = y_ref[0]
    mn = mn_ref[0, 0, :]
    mx = mx_ref[0, 0, :]
    o_ref[0] = (yb - mn[:, None, None]) / (mx - mn + 1e-8)[:, None, None]


@jax.jit
def kernel(x):
    f32 = jnp.float32

    sums = pl.pallas_call(
        _sums_body,
        grid=(_B, _NH),
        in_specs=[pl.BlockSpec((1, _C, _HB, _W), lambda b, h: (b, 0, h, 0))],
        out_specs=pl.BlockSpec((1, 1, _C), lambda b, h: (b, 0, 0)),
        out_shape=jax.ShapeDtypeStruct((_B, 1, _C), f32),
    )(x)

    weights = pl.pallas_call(
        _weights_body,
        in_specs=[pl.BlockSpec((_B, 1, _C), lambda: (0, 0, 0))],
        out_specs=pl.BlockSpec((_B, _K, _C), lambda: (0, 0, 0)),
        out_shape=jax.ShapeDtypeStruct((_B, _K, _C), f32),
    )(sums)

    y, mn, mx = pl.pallas_call(
        _rcp_body,
        grid=(_B, _NH),
        in_specs=[
            pl.BlockSpec((1, _C, _HB, _W), lambda b, h: (b, 0, h, 0)),
            pl.BlockSpec((1, _K, _C), lambda b, h: (b, 0, 0)),
        ],
        out_specs=[
            pl.BlockSpec((1, _K, _HB, _W), lambda b, h: (b, 0, h, 0)),
            pl.BlockSpec((1, 1, _K), lambda b, h: (b, 0, 0)),
            pl.BlockSpec((1, 1, _K), lambda b, h: (b, 0, 0)),
        ],
        out_shape=[
            jax.ShapeDtypeStruct((_B, _K, _H, _W), f32),
            jax.ShapeDtypeStruct((_B, 1, _K), f32),
            jax.ShapeDtypeStruct((_B, 1, _K), f32),
        ],
    )(x, weights)

    out = pl.pallas_call(
        _norm_body,
        grid=(_B, _NH),
        in_specs=[
            pl.BlockSpec((1, _K, _HB, _W), lambda b, h: (b, 0, h, 0)),
            pl.BlockSpec((1, 1, _K), lambda b, h: (b, 0, 0)),
            pl.BlockSpec((1, 1, _K), lambda b, h: (b, 0, 0)),
        ],
        out_specs=pl.BlockSpec((1, _K, _HB, _W), lambda b, h: (b, 0, h, 0)),
        out_shape=jax.ShapeDtypeStruct((_B, _K, _H, _W), f32),
    )(y, mn, mx)

    return out


# speedup vs baseline: 63.7652x; 1.1747x over previous
import functools

import jax
import jax.numpy as jnp
from jax.experimental import pallas as pl
from jax.experimental.pallas import tpu as pltpu

_B, _C, _H, _W = 4, 64, 384, 384
_HB = 48
_NH = _H // _HB
_SPLITS = (16, 32, 48)
_K = 2 * len(_SPLITS)
_RB = 8
_Q = _C // 4


def _sums_body(x_ref, s_ref):
    h = pl.program_id(1)
    part = jnp.sum(x_ref[0], axis=(1, 2))

    @pl.when(h == 0)
    def _():
        s_ref[0, 0, :] = part

    @pl.when(h != 0)
    def _():
        s_ref[0, 0, :] = s_ref[0, 0, :] + part


def _perm_body(s_ref, p_ref):
    m = s_ref[:, 0, :]
    mc = m[:, :, None]
    mj = m[:, None, :]
    jj = jax.lax.broadcasted_iota(jnp.int32, (_B, _C, _C), 2)
    cc = jax.lax.broadcasted_iota(jnp.int32, (_B, _C, _C), 1)
    cnt = jnp.where((mj < mc) | ((mj == mc) & (jj < cc)), 1, 0)
    rank = jnp.sum(cnt, axis=2)
    rr = jax.lax.broadcasted_iota(jnp.int32, (_B, _C, _C), 1)
    ci = jax.lax.broadcasted_iota(jnp.int32, (_B, _C, _C), 2)
    perm = jnp.sum(jnp.where(rank[:, None, :] == rr, ci, 0), axis=2)
    p_ref[:, 0, :] = perm


_C16_S, _C16_G = 1.0 / 48, -(1.0 / 16 + 1.0 / 48)
_C32_S, _C32_G = 1.0 / 32, -(1.0 / 16)
_C48_S, _C48_G = -(1.0 / 48), (1.0 / 48 + 1.0 / 16)


def _rcp_body(p_ref, x_ref, y_ref, mn_ref, mx_ref):
    b = pl.program_id(0)
    h = pl.program_id(1)
    vmins, vmaxs = [], []
    for hs in range(0, _HB, _RB):
        q = [jnp.zeros((_RB, _W), jnp.float32) for _ in range(4)]
        g = [jnp.zeros((_RB, _W), jnp.float32) for _ in range(4)]
        for c in range(_C):
            q[c // _Q] = q[c // _Q] + x_ref[0, c, hs:hs + _RB, :]
        for r in range(_C):
            pc = p_ref[b, 0, r]
            g[r // _Q] = g[r // _Q] + x_ref[0, pc, hs:hs + _RB, :]
        s = (q[0] + q[1]) + (q[2] + q[3])
        g01 = g[0] + g[1]
        q01 = q[0] + q[1]
        maps = [
            _C16_S * s + _C16_G * g[0],
            _C32_S * s + _C32_G * g01,
            _C48_S * s + _C48_G * g[3],
            _C16_S * s + _C16_G * q[0],
            _C32_S * s + _C32_G * q01,
            _C48_S * s + _C48_G * q[3],
        ]
        for k in range(_K):
            y_ref[0, k, hs:hs + _RB, :] = maps[k]
        vmins.append(jnp.stack([jnp.min(a) for a in maps]))
        vmaxs.append(jnp.stack([jnp.max(a) for a in maps]))
    vmin = functools.reduce(jnp.minimum, vmins)
    vmax = functools.reduce(jnp.maximum, vmaxs)

    @pl.when(h == 0)
    def _():
        mn_ref[0, 0, :] = vmin
        mx_ref[0, 0, :] = vmax

    @pl.when(h != 0)
    def _():
        mn_ref[0, 0, :] = jnp.minimum(mn_ref[0, 0, :], vmin)
        mx_ref[0, 0, :] = jnp.maximum(mx_ref[0, 0, :], vmax)


def _norm_body(y_ref, mn_ref, mx_ref, o_ref):
    yb = y_ref[0]
    mn = mn_ref[0, 0, :]
    mx = mx_ref[0, 0, :]
    o_ref[0] = (yb - mn[:, None, None]) / (mx - mn + 1e-8)[:, None, None]


@jax.jit
def kernel(x):
    f32 = jnp.float32

    sums = pl.pallas_call(
        _sums_body,
        grid=(_B, _NH),
        in_specs=[pl.BlockSpec((1, _C, _HB, _W), lambda b, h: (b, 0, h, 0))],
        out_specs=pl.BlockSpec((1, 1, _C), lambda b, h: (b, 0, 0)),
        out_shape=jax.ShapeDtypeStruct((_B, 1, _C), f32),
    )(x)

    perm = pl.pallas_call(
        _perm_body,
        in_specs=[pl.BlockSpec((_B, 1, _C), lambda: (0, 0, 0))],
        out_specs=pl.BlockSpec((_B, 1, _C), lambda: (0, 0, 0)),
        out_shape=jax.ShapeDtypeStruct((_B, 1, _C), jnp.int32),
    )(sums)

    y, mn, mx = pl.pallas_call(
        _rcp_body,
        grid=(_B, _NH),
        in_specs=[
            pl.BlockSpec(memory_space=pltpu.SMEM),
            pl.BlockSpec((1, _C, _HB, _W), lambda b, h: (b, 0, h, 0)),
        ],
        out_specs=[
            pl.BlockSpec((1, _K, _HB, _W), lambda b, h: (b, 0, h, 0)),
            pl.BlockSpec((1, 1, _K), lambda b, h: (b, 0, 0)),
            pl.BlockSpec((1, 1, _K), lambda b, h: (b, 0, 0)),
        ],
        out_shape=[
            jax.ShapeDtypeStruct((_B, _K, _H, _W), f32),
            jax.ShapeDtypeStruct((_B, 1, _K), f32),
            jax.ShapeDtypeStruct((_B, 1, _K), f32),
        ],
    )(perm, x)

    out = pl.pallas_call(
        _norm_body,
        grid=(_B, _NH),
        in_specs=[
            pl.BlockSpec((1, _K, _HB, _W), lambda b, h: (b, 0, h, 0)),
            pl.BlockSpec((1, 1, _K), lambda b, h: (b, 0, 0)),
            pl.BlockSpec((1, 1, _K), lambda b, h: (b, 0, 0)),
        ],
        out_specs=pl.BlockSpec((1, _K, _HB, _W), lambda b, h: (b, 0, h, 0)),
        out_shape=jax.ShapeDtypeStruct((_B, _K, _H, _W), f32),
    )(y, mn, mx)

    return out
